# baseline (speedup 1.0000x reference)
import functools

import jax
import jax.numpy as jnp
from jax import lax
from jax.experimental import pallas as pl
from jax.experimental.pallas import tpu as pltpu
from jax.experimental.pallas import tpu_sc as plsc

NUM_CLASSES = 1000000
EMBED_DIM = 64
BATCH = 16384

_INFO = plsc.get_sparse_core_info()
_NC = _INFO.num_cores
_NS = _INFO.num_subcores
_NW = _NC * _NS
_L = _INFO.num_lanes

_B_PER_W = BATCH // _NW
_CHUNK = 128
_NCHUNK = _B_PER_W // _CHUNK
_GROUPS = _B_PER_W // _L


def _sc_kernel(x_idx_hbm, y_idx_hbm, table_hbm, out_hbm,
               xidx_v, yidx_v, xrows_v, yrows_v, out_v, sem):
    wid = lax.axis_index("s") * _NC + lax.axis_index("c")

    pltpu.sync_copy(x_idx_hbm.at[wid], xidx_v)
    pltpu.sync_copy(y_idx_hbm.at[wid], yidx_v)

    copies = []
    for c in range(_NCHUNK):
        copies.append(pltpu.async_copy(
            table_hbm.at[xidx_v.at[c]],
            xrows_v.at[pl.ds(c * _CHUNK, _CHUNK)], sem))
        copies.append(pltpu.async_copy(
            table_hbm.at[yidx_v.at[c]],
            yrows_v.at[pl.ds(c * _CHUNK, _CHUNK)], sem))
    for cp in copies:
        cp.wait()

    iota = lax.broadcasted_iota(jnp.int32, (_L,), 0)

    def group_body(g, carry):
        rowv = g * _L + iota
        acc = jnp.zeros((_L,), jnp.float32)
        for j in range(EMBED_DIM):
            colv = jnp.full((_L,), j, jnp.int32)
            gx = plsc.load_gather(xrows_v, [rowv, colv])
            gy = plsc.load_gather(yrows_v, [rowv, colv])
            acc = acc + gx * gy
        out_v[pl.ds(g * _L, _L)] = 1.0 / (1.0 + jnp.exp(acc))
        return carry

    lax.fori_loop(0, _GROUPS, group_body, 0)

    pltpu.sync_copy(out_v, out_hbm.at[pl.ds(wid * _B_PER_W, _B_PER_W)])


@jax.jit
def kernel(table, x_idx, y_idx):
    x3 = x_idx.reshape(_NW, _NCHUNK, _CHUNK)
    y3 = y_idx.reshape(_NW, _NCHUNK, _CHUNK)
    mesh = plsc.VectorSubcoreMesh(core_axis_name="c", subcore_axis_name="s")
    run = functools.partial(
        pl.kernel, mesh=mesh,
        compiler_params=pltpu.CompilerParams(
            needs_layout_passes=False, use_tc_tiling_on_sc=False),
        out_type=jax.ShapeDtypeStruct((BATCH,), jnp.float32),
        scratch_types=[
            pltpu.VMEM((_NCHUNK, _CHUNK), jnp.int32),
            pltpu.VMEM((_NCHUNK, _CHUNK), jnp.int32),
            pltpu.VMEM((_B_PER_W, EMBED_DIM), jnp.float32),
            pltpu.VMEM((_B_PER_W, EMBED_DIM), jnp.float32),
            pltpu.VMEM((_B_PER_W,), jnp.float32),
            pltpu.SemaphoreType.DMA,
        ],
    )(_sc_kernel)
    return run(x3, y3, table)
